# baseline (speedup 1.0000x reference)
import dataclasses
import functools

import jax
import jax.numpy as jnp
from jax import lax
from jax.experimental import pallas as pl
from jax.experimental.pallas import tpu as pltpu
from jax.experimental.pallas import tpu_sc as plsc

N_EMBED = 16
BATCH = 16384
NUM_CORES = 2
NUM_SUBCORES = 16
NUM_WORKERS = NUM_CORES * NUM_SUBCORES
B_PER_W = BATCH // NUM_WORKERS
LANES = 16
N_ROWS = 1000000
PLANE = ((N_ROWS + 127) // 128) * 1024
FLAT = N_EMBED * N_ROWS


def _sc_embed_dot(u_idx, i_idx, mu, bias, wu_t, wi_t):
    mesh = plsc.VectorSubcoreMesh(core_axis_name="c", subcore_axis_name="s")

    cp = pltpu.CompilerParams()
    fields = pltpu.CompilerParams.__dataclass_fields__
    if "needs_layout_passes" in fields:
        cp = dataclasses.replace(cp, needs_layout_passes=False)
    if "disable_bounds_checks" in fields:
        cp = dataclasses.replace(cp, disable_bounds_checks=True)

    @functools.partial(
        pl.kernel,
        compiler_params=cp,
        out_type=jax.ShapeDtypeStruct((BATCH,), jnp.float32),
        mesh=mesh,
        scratch_types=[
            pltpu.VMEM((B_PER_W,), jnp.int32),
            pltpu.VMEM((B_PER_W,), jnp.int32),
            pltpu.VMEM((N_EMBED * B_PER_W,), jnp.int32),
            pltpu.VMEM((N_EMBED * B_PER_W,), jnp.int32),
            pltpu.VMEM((N_EMBED * B_PER_W,), jnp.float32),
            pltpu.VMEM((N_EMBED * B_PER_W,), jnp.float32),
            pltpu.VMEM((B_PER_W,), jnp.float32),
            pltpu.VMEM((B_PER_W,), jnp.float32),
            pltpu.VMEM((B_PER_W,), jnp.float32),
            pltpu.SemaphoreType.DMA,
        ],
    )
    def k(u_hbm, i_hbm, mu_hbm, b_hbm, wu_hbm, wi_hbm, out_hbm,
          uidx_v, iidx_v, pu_v, pi_v, uval_v, ival_v, mu_v, b_v, out_v,
          sem):
        wid = lax.axis_index("s") * NUM_CORES + lax.axis_index("c")
        base = wid * B_PER_W
        sl = pl.ds(base, B_PER_W)

        pltpu.sync_copy(u_hbm.at[sl], uidx_v)
        pltpu.sync_copy(i_hbm.at[sl], iidx_v)

        @pl.loop(0, B_PER_W, step=LANES)
        def _(c):
            u16 = uidx_v[pl.ds(c, LANES)]
            i16 = iidx_v[pl.ds(c, LANES)]
            for e in range(N_EMBED):
                off = e * N_ROWS
                pu_v[pl.ds(e * B_PER_W + c, LANES)] = u16 + off
                pi_v[pl.ds(e * B_PER_W + c, LANES)] = i16 + off

        cp_u = pltpu.async_copy(wu_hbm.at[pu_v], uval_v, sem)
        cp_i = pltpu.async_copy(wi_hbm.at[pi_v], ival_v, sem)
        pltpu.sync_copy(mu_hbm.at[sl], mu_v)
        pltpu.sync_copy(b_hbm.at[sl], b_v)
        cp_u.wait()
        cp_i.wait()

        @pl.loop(0, B_PER_W, step=LANES)
        def _(c):
            acc = mu_v[pl.ds(c, LANES)] + 2.0 * b_v[pl.ds(c, LANES)]
            for e in range(N_EMBED):
                uv = uval_v[pl.ds(e * B_PER_W + c, LANES)]
                iv = ival_v[pl.ds(e * B_PER_W + c, LANES)]
                acc = acc + uv * iv
            out_v[pl.ds(c, LANES)] = acc

        pltpu.sync_copy(out_v, out_hbm.at[sl])

    return k(u_idx, i_idx, mu, bias, wu_t, wi_t)


def kernel(x, W_user, W_item):
    u_idx = x[:, 0].astype(jnp.int32)
    i_idx = x[:, 1].astype(jnp.int32)
    mu = x[:, 2]
    bias = x[:, 3]
    wu_flat = W_user.T.reshape(-1)
    wi_flat = W_item.T.reshape(-1)
    return _sc_embed_dot(u_idx, i_idx, mu, bias, wu_flat, wi_flat)


# speedup vs baseline: 20.2311x; 20.2311x over previous
import dataclasses
import functools

import jax
import jax.numpy as jnp
from jax import lax
from jax.experimental import pallas as pl
from jax.experimental.pallas import tpu as pltpu
from jax.experimental.pallas import tpu_sc as plsc

N_EMBED = 16
BATCH = 16384
NUM_CORES = 2
NUM_SUBCORES = 16
NUM_WORKERS = NUM_CORES * NUM_SUBCORES
B_PER_W = BATCH // NUM_WORKERS
LANES = 16
N_ROWS = 1000000
N_TILE_COLS = (N_ROWS + 127) // 128
PLANE = N_TILE_COLS * 1024
N_LINES = 2 * N_TILE_COLS * 8
FLAT = N_LINES * 128
CHUNK_COLS = 76928
LINES_PER_STEP = CHUNK_COLS // 16


def _tc_repack(wt):
    def body(x_ref, o_ref):
        o_ref[...] = pltpu.einshape("a(bc)->(ba)c", x_ref[...], c=128)

    return pl.pallas_call(
        body,
        grid=(2, 13),
        in_specs=[pl.BlockSpec((8, CHUNK_COLS), lambda e, i: (e, i))],
        out_specs=pl.BlockSpec(
            (LINES_PER_STEP, 128), lambda e, i: (e * 13 + i, 0)
        ),
        out_shape=jax.ShapeDtypeStruct((N_LINES, 128), jnp.float32),
        compiler_params=pltpu.CompilerParams(
            dimension_semantics=("parallel", "arbitrary"),
        ),
    )(wt)


def _sc_embed_dot(u_idx, i_idx, mu, bias, wu_flat, wi_flat):
    mesh = plsc.VectorSubcoreMesh(core_axis_name="c", subcore_axis_name="s")

    cp = pltpu.CompilerParams()
    fields = pltpu.CompilerParams.__dataclass_fields__
    if "needs_layout_passes" in fields:
        cp = dataclasses.replace(cp, needs_layout_passes=False)

    @functools.partial(
        pl.kernel,
        compiler_params=cp,
        out_type=jax.ShapeDtypeStruct((BATCH,), jnp.float32),
        mesh=mesh,
        scratch_types=[
            pltpu.VMEM((B_PER_W,), jnp.int32),
            pltpu.VMEM((B_PER_W,), jnp.int32),
            pltpu.VMEM((N_EMBED * B_PER_W,), jnp.int32),
            pltpu.VMEM((N_EMBED * B_PER_W,), jnp.int32),
            pltpu.VMEM((N_EMBED * B_PER_W,), jnp.float32),
            pltpu.VMEM((N_EMBED * B_PER_W,), jnp.float32),
            pltpu.VMEM((B_PER_W,), jnp.float32),
            pltpu.VMEM((B_PER_W,), jnp.float32),
            pltpu.VMEM((B_PER_W,), jnp.float32),
            pltpu.SemaphoreType.DMA,
        ],
    )
    def k(u_hbm, i_hbm, mu_hbm, b_hbm, wu_hbm, wi_hbm, out_hbm,
          uidx_v, iidx_v, pu_v, pi_v, uval_v, ival_v, mu_v, b_v, out_v,
          sem):
        wid = lax.axis_index("s") * NUM_CORES + lax.axis_index("c")
        base = wid * B_PER_W
        sl = pl.ds(base, B_PER_W)

        pltpu.sync_copy(u_hbm.at[sl], uidx_v)
        pltpu.sync_copy(i_hbm.at[sl], iidx_v)

        @pl.loop(0, B_PER_W, step=LANES)
        def _(c):
            u16 = uidx_v[pl.ds(c, LANES)]
            i16 = iidx_v[pl.ds(c, LANES)]
            ub = ((u16 >> 7) << 10) + (u16 & 127)
            ib = ((i16 >> 7) << 10) + (i16 & 127)
            for e in range(N_EMBED):
                off = (e // 8) * PLANE + (e % 8) * 128
                pu_v[pl.ds(e * B_PER_W + c, LANES)] = ub + off
                pi_v[pl.ds(e * B_PER_W + c, LANES)] = ib + off

        cp_u = pltpu.async_copy(wu_hbm.at[pu_v], uval_v, sem)
        cp_i = pltpu.async_copy(wi_hbm.at[pi_v], ival_v, sem)
        pltpu.sync_copy(mu_hbm.at[sl], mu_v)
        pltpu.sync_copy(b_hbm.at[sl], b_v)
        cp_u.wait()
        cp_i.wait()

        @pl.loop(0, B_PER_W, step=LANES)
        def _(c):
            acc = mu_v[pl.ds(c, LANES)] + 2.0 * b_v[pl.ds(c, LANES)]
            for e in range(N_EMBED):
                uv = uval_v[pl.ds(e * B_PER_W + c, LANES)]
                iv = ival_v[pl.ds(e * B_PER_W + c, LANES)]
                acc = acc + uv * iv
            out_v[pl.ds(c, LANES)] = acc

        pltpu.sync_copy(out_v, out_hbm.at[sl])

    return k(u_idx, i_idx, mu, bias, wu_flat, wi_flat)


def kernel(x, W_user, W_item):
    u_idx = x[:, 0].astype(jnp.int32)
    i_idx = x[:, 1].astype(jnp.int32)
    mu = x[:, 2]
    bias = x[:, 3]
    wu_flat = _tc_repack(W_user.T).reshape(-1)
    wi_flat = _tc_repack(W_item.T).reshape(-1)
    return _sc_embed_dot(u_idx, i_idx, mu, bias, wu_flat, wi_flat)


# speedup vs baseline: 21.3279x; 1.0542x over previous
import dataclasses
import functools

import jax
import jax.numpy as jnp
from jax import lax
from jax.experimental import pallas as pl
from jax.experimental.pallas import tpu as pltpu
from jax.experimental.pallas import tpu_sc as plsc

N_EMBED = 16
BATCH = 16384
NUM_CORES = 2
NUM_SUBCORES = 16
NUM_WORKERS = NUM_CORES * NUM_SUBCORES
B_PER_W = BATCH // NUM_WORKERS
LANES = 16
N_ROWS = 1000000
N_TILE_COLS = (N_ROWS + 127) // 128
PLANE = N_TILE_COLS * 1024
N_LINES = 2 * N_TILE_COLS * 8
FLAT = N_LINES * 128
CHUNK_COLS = 76928
LINES_PER_STEP = CHUNK_COLS // 16


def _tc_repack(wu_t, wi_t):
    def body(u_ref, i_ref, ou_ref, oi_ref):
        ou_ref[...] = pltpu.einshape("a(bc)->(ba)c", u_ref[...], c=128)
        oi_ref[...] = pltpu.einshape("a(bc)->(ba)c", i_ref[...], c=128)

    in_spec = pl.BlockSpec((8, CHUNK_COLS), lambda e, i: (e, i))
    out_spec = pl.BlockSpec(
        (LINES_PER_STEP, 128), lambda e, i: (e * 13 + i, 0)
    )
    return pl.pallas_call(
        body,
        grid=(2, 13),
        in_specs=[in_spec, in_spec],
        out_specs=[out_spec, out_spec],
        out_shape=[
            jax.ShapeDtypeStruct((N_LINES, 128), jnp.float32),
            jax.ShapeDtypeStruct((N_LINES, 128), jnp.float32),
        ],
        compiler_params=pltpu.CompilerParams(
            dimension_semantics=("parallel", "arbitrary"),
        ),
    )(wu_t, wi_t)


def _sc_embed_dot(u_idx, i_idx, mu, bias, wu_flat, wi_flat):
    mesh = plsc.VectorSubcoreMesh(core_axis_name="c", subcore_axis_name="s")

    cp = pltpu.CompilerParams()
    fields = pltpu.CompilerParams.__dataclass_fields__
    if "needs_layout_passes" in fields:
        cp = dataclasses.replace(cp, needs_layout_passes=False)

    @functools.partial(
        pl.kernel,
        compiler_params=cp,
        out_type=jax.ShapeDtypeStruct((BATCH,), jnp.float32),
        mesh=mesh,
        scratch_types=[
            pltpu.VMEM((B_PER_W,), jnp.int32),
            pltpu.VMEM((B_PER_W,), jnp.int32),
            pltpu.VMEM((N_EMBED * B_PER_W,), jnp.int32),
            pltpu.VMEM((N_EMBED * B_PER_W,), jnp.int32),
            pltpu.VMEM((N_EMBED * B_PER_W,), jnp.float32),
            pltpu.VMEM((N_EMBED * B_PER_W,), jnp.float32),
            pltpu.VMEM((B_PER_W,), jnp.float32),
            pltpu.VMEM((B_PER_W,), jnp.float32),
            pltpu.VMEM((B_PER_W,), jnp.float32),
            pltpu.SemaphoreType.DMA,
        ],
    )
    def k(u_hbm, i_hbm, mu_hbm, b_hbm, wu_hbm, wi_hbm, out_hbm,
          uidx_v, iidx_v, pu_v, pi_v, uval_v, ival_v, mu_v, b_v, out_v,
          sem):
        wid = lax.axis_index("s") * NUM_CORES + lax.axis_index("c")
        base = wid * B_PER_W
        sl = pl.ds(base, B_PER_W)

        pltpu.sync_copy(u_hbm.at[sl], uidx_v)
        pltpu.sync_copy(i_hbm.at[sl], iidx_v)

        @pl.loop(0, B_PER_W, step=LANES)
        def _(c):
            u16 = uidx_v[pl.ds(c, LANES)]
            ub = ((u16 >> 7) << 10) + (u16 & 127)
            for e in range(N_EMBED):
                off = (e // 8) * PLANE + (e % 8) * 128
                pu_v[pl.ds(e * B_PER_W + c, LANES)] = ub + off

        cp_u = pltpu.async_copy(wu_hbm.at[pu_v], uval_v, sem)

        @pl.loop(0, B_PER_W, step=LANES)
        def _(c):
            i16 = iidx_v[pl.ds(c, LANES)]
            ib = ((i16 >> 7) << 10) + (i16 & 127)
            for e in range(N_EMBED):
                off = (e // 8) * PLANE + (e % 8) * 128
                pi_v[pl.ds(e * B_PER_W + c, LANES)] = ib + off

        cp_i = pltpu.async_copy(wi_hbm.at[pi_v], ival_v, sem)
        pltpu.sync_copy(mu_hbm.at[sl], mu_v)
        pltpu.sync_copy(b_hbm.at[sl], b_v)
        cp_u.wait()
        cp_i.wait()

        @pl.loop(0, B_PER_W, step=LANES)
        def _(c):
            acc = mu_v[pl.ds(c, LANES)] + 2.0 * b_v[pl.ds(c, LANES)]
            for e in range(N_EMBED):
                uv = uval_v[pl.ds(e * B_PER_W + c, LANES)]
                iv = ival_v[pl.ds(e * B_PER_W + c, LANES)]
                acc = acc + uv * iv
            out_v[pl.ds(c, LANES)] = acc

        pltpu.sync_copy(out_v, out_hbm.at[sl])

    return k(u_idx, i_idx, mu, bias, wu_flat, wi_flat)


def kernel(x, W_user, W_item):
    u_idx = x[:, 0].astype(jnp.int32)
    i_idx = x[:, 1].astype(jnp.int32)
    mu = x[:, 2]
    bias = x[:, 3]
    wu_c, wi_c = _tc_repack(W_user.T, W_item.T)
    return _sc_embed_dot(u_idx, i_idx, mu, bias,
                         wu_c.reshape(-1), wi_c.reshape(-1))
